# baseline (speedup 1.0000x reference)
import functools

import jax
import jax.numpy as jnp
from jax import lax
from jax.experimental import pallas as pl
from jax.experimental.pallas import tpu as pltpu
from jax.experimental.pallas import tpu_sc as plsc

N = 10000
E = 320000
D = 128

NC = 2
NS = 16
NW = NC * NS

C = 112
NCH = (24, 156)
NCHUNK_TOT = NS * (NCH[0] + NCH[1])
E_PAD = NCHUNK_TOT * C
N_PAD = N + 8
ROWS_PER_TILE = 624


def _sc_agg_body(with_deg, *refs):
    if with_deg:
        (h_hbm, src_hbm, dst_hbm, z2_hbm, z1_hbm, parts_hbm, degp_hbm,
         s0, s1, s2, s3, s4, s5, d0_, d1_, d2_, d3_, d4_, d5_,
         ones_v, zbuf, acc, deg_acc,
         rows0, rows1, rows2,
         gs0, gs1, gs2, ss0, ss1, ss2,
         is0, is1, is2, is3, is4, is5) = refs
    else:
        (h_hbm, src_hbm, dst_hbm, z2_hbm, parts_hbm,
         s0, s1, s2, s3, s4, s5, d0_, d1_, d2_, d3_, d4_, d5_, acc,
         rows0, rows1, rows2,
         gs0, gs1, gs2, ss0, ss1, ss2,
         is0, is1, is2, is3, is4, is5) = refs
    rows = (rows0, rows1, rows2)
    sidx = (s0, s1, s2, s3, s4, s5)
    didx = (d0_, d1_, d2_, d3_, d4_, d5_)
    gsem = (gs0, gs1, gs2)
    ssem = (ss0, ss1, ss2)
    isem = (is0, is1, is2, is3, is4, is5)

    cid = lax.axis_index("c")
    sid = lax.axis_index("s")
    nchunk = jnp.where(cid == 0, NCH[0], NCH[1])

    r0 = pl.multiple_of(sid * ROWS_PER_TILE, 8)
    pltpu.sync_copy(z2_hbm.at[pl.ds(r0, ROWS_PER_TILE)],
                    acc.at[pl.ds(r0, ROWS_PER_TILE)])

    @pl.when(sid == 0)
    def _zero_rest():
        pltpu.sync_copy(z2_hbm.at[pl.ds(NS * ROWS_PER_TILE, N_PAD - NS * ROWS_PER_TILE)],
                        acc.at[pl.ds(NS * ROWS_PER_TILE, N_PAD - NS * ROWS_PER_TILE)])

    if with_deg:
        pltpu.sync_copy(z1_hbm.at[pl.ds(r0, ROWS_PER_TILE)], zbuf)
        pltpu.sync_copy(zbuf, deg_acc.at[pl.ds(r0, ROWS_PER_TILE)])

        @pl.when(sid == 0)
        def _zero_deg_rest():
            rr = pl.multiple_of(NS * ROWS_PER_TILE, 8)
            nrem = N_PAD - NS * ROWS_PER_TILE
            pltpu.sync_copy(z1_hbm.at[pl.ds(rr, nrem)], zbuf.at[pl.ds(0, nrem)])
            pltpu.sync_copy(zbuf.at[pl.ds(0, nrem)], deg_acc.at[pl.ds(rr, nrem)])

        for k in range(C // 16):
            ones_v[pl.ds(k * 16, 16)] = jnp.ones((16,), jnp.float32)

    plsc.subcore_barrier()

    base = jnp.where(cid == 0, sid * (NCH[0] * C),
                     NS * NCH[0] * C + sid * (NCH[1] * C))

    def load_idx(c, k, sfx):
        off = pl.multiple_of(sfx + c * C, 8)
        pltpu.async_copy(src_hbm.at[pl.ds(off, C)], sidx[k], isem[k])
        pltpu.async_copy(dst_hbm.at[pl.ds(off, C)], didx[k], isem[k])

    def wait_idx(c, k, sfx):
        off = pl.multiple_of(sfx + c * C, 8)
        pltpu.make_async_copy(src_hbm.at[pl.ds(off, C)], sidx[k],
                              isem[k]).wait()
        pltpu.make_async_copy(dst_hbm.at[pl.ds(off, C)], didx[k],
                              isem[k]).wait()

    for k in range(6):
        load_idx(k, k, base)
    for b in range(2):
        wait_idx(b, b, base)
        pltpu.async_copy(h_hbm.at[sidx[b]], rows[b], gsem[b])

    def step6(j, carry):
        c0 = j * 6
        for u in range(6):
            v = u % 3
            c = c0 + u
            pltpu.make_async_copy(h_hbm.at[sidx[u]], rows[v],
                                  gsem[v]).wait()
            pltpu.async_copy(rows[v], acc.at[didx[u]], ssem[v], add=True)
            if with_deg:
                pltpu.sync_copy(ones_v, deg_acc.at[didx[u]], add=True)

            @pl.when(c >= 1)
            def _drain_prev_scatter():
                vp = (v + 2) % 3
                pltpu.make_async_copy(rows[vp], acc.at[didx[(u + 5) % 6]],
                                      ssem[vp]).wait()

            @pl.when((c >= 1) & (c + 5 < nchunk))
            def _prefetch_idx():
                load_idx(c + 5, (u + 5) % 6, base)

            @pl.when(c + 2 < nchunk)
            def _next_gather():
                k2 = (u + 2) % 6
                v2 = (v + 2) % 3
                wait_idx(c + 2, k2, base)
                pltpu.async_copy(h_hbm.at[sidx[k2]], rows[v2], gsem[v2])
        return carry

    lax.fori_loop(0, nchunk // 6, step6, 0)

    pltpu.make_async_copy(rows[2], acc.at[didx[5]], ssem[2]).wait()

    plsc.subcore_barrier()

    pltpu.sync_copy(acc.at[pl.ds(r0, ROWS_PER_TILE)],
                    parts_hbm.at[cid, pl.ds(r0, ROWS_PER_TILE), :])

    @pl.when(sid == 0)
    def _out_rest():
        rr = pl.multiple_of(NS * ROWS_PER_TILE, 8)
        pltpu.sync_copy(acc.at[pl.ds(rr, N - NS * ROWS_PER_TILE)],
                        parts_hbm.at[cid, pl.ds(rr, N - NS * ROWS_PER_TILE), :])

    if with_deg:
        d0 = pl.multiple_of(cid * N + r0, 8)
        pltpu.sync_copy(deg_acc.at[pl.ds(r0, ROWS_PER_TILE)], zbuf)
        pltpu.sync_copy(zbuf, degp_hbm.at[pl.ds(d0, ROWS_PER_TILE)])

        @pl.when(sid == 0)
        def _deg_out_rest():
            rr = pl.multiple_of(NS * ROWS_PER_TILE, 8)
            dd = pl.multiple_of(cid * N + rr, 8)
            nrem = N - NS * ROWS_PER_TILE
            pltpu.sync_copy(deg_acc.at[pl.ds(rr, nrem)], zbuf.at[pl.ds(0, nrem)])
            pltpu.sync_copy(zbuf.at[pl.ds(0, nrem)], degp_hbm.at[pl.ds(dd, nrem)])


def _make_sc_agg(with_deg):
    mesh = plsc.VectorSubcoreMesh(core_axis_name="c", subcore_axis_name="s",
                                  num_cores=NC, num_subcores=NS)
    out_type = [jax.ShapeDtypeStruct((NC, N, D), jnp.float32)]
    if with_deg:
        out_type.append(jax.ShapeDtypeStruct((NC * N,), jnp.float32))
    scratch = [pltpu.VMEM((C,), jnp.int32) for _ in range(12)]
    if with_deg:
        scratch.append(pltpu.VMEM((C,), jnp.float32))
        scratch.append(pltpu.VMEM((ROWS_PER_TILE,), jnp.float32))
    scratch.append(pltpu.VMEM_SHARED((N_PAD, D), jnp.float32))
    if with_deg:
        scratch.append(pltpu.VMEM_SHARED((N_PAD,), jnp.float32))
    scratch.extend(pltpu.VMEM((C, D), jnp.float32) for _ in range(3))
    scratch.extend(pltpu.SemaphoreType.DMA for _ in range(12))
    return pl.kernel(functools.partial(_sc_agg_body, with_deg),
                     out_type=tuple(out_type), mesh=mesh,
                     scratch_types=tuple(scratch))


_sc_agg_first = _make_sc_agg(True)
_sc_agg_rest = _make_sc_agg(False)


def _combine_body(relu, parts_ref, degp_ref, h_ref, wv_ref, b_ref, out_ref):
    agg = parts_ref[0] + parts_ref[1]
    deg = degp_ref[0] + degp_ref[1]
    aggn = agg / jnp.clip(deg, 1.0, None)
    cat = jnp.concatenate([aggn, h_ref[...]], axis=1)
    out = jax.lax.dot_general(cat, wv_ref[...], (((1,), (0,)), ((), ())),
                              preferred_element_type=jnp.float32)
    out = out + b_ref[...]
    if relu:
        out = jnp.maximum(out, 0.0)
    out_ref[...] = out


R = 1000


def _make_combine(relu):
    return pl.pallas_call(
        functools.partial(_combine_body, relu),
        grid=(N // R,),
        in_specs=[
            pl.BlockSpec((NC, R, D), lambda i: (0, i, 0)),
            pl.BlockSpec((NC, R, 1), lambda i: (0, i, 0)),
            pl.BlockSpec((R, D), lambda i: (i, 0)),
            pl.BlockSpec((2 * D, D), lambda i: (0, 0)),
            pl.BlockSpec((1, D), lambda i: (0, 0)),
        ],
        out_specs=pl.BlockSpec((R, D), lambda i: (i, 0)),
        out_shape=jax.ShapeDtypeStruct((N, D), jnp.float32),
    )


_combine_relu = _make_combine(True)
_combine_lin = _make_combine(False)


def kernel(x, edge_index, Wl1, bl1, Wr1, Wl2, bl2, Wr2, Wl3, bl3, Wr3,
           Wl4, bl4, Wr4):
    src = edge_index[0].astype(jnp.int32)
    dst = edge_index[1].astype(jnp.int32)
    pad = E_PAD - E
    src_p = jnp.concatenate([src, jnp.zeros((pad,), jnp.int32)])
    dst_p = jnp.concatenate(
        [dst, N + (jnp.arange(pad, dtype=jnp.int32) % (N_PAD - N))])
    z2 = jnp.zeros((N_PAD, D), jnp.float32)
    z1 = jnp.zeros((N_PAD,), jnp.float32)

    parts, degp = _sc_agg_first(x, src_p, dst_p, z2, z1)
    degp3 = degp.reshape(NC, N, 1)

    layers = [(Wl1, bl1, Wr1, True), (Wl2, bl2, Wr2, False),
              (Wl3, bl3, Wr3, False), (Wl4, bl4, Wr4, False)]

    h = x
    for li, (Wl, bl, Wr, relu) in enumerate(layers):
        if li > 0:
            (parts,) = _sc_agg_rest(h, src_p, dst_p, z2)
        wv = jnp.concatenate([Wl.T, Wr.T], axis=0)
        comb = _combine_relu if relu else _combine_lin
        h = comb(parts, degp3, h, wv, bl.reshape(1, D))
    return h


# speedup vs baseline: 1.4083x; 1.4083x over previous
import functools

import jax
import jax.numpy as jnp
from jax import lax
from jax.experimental import pallas as pl
from jax.experimental.pallas import tpu as pltpu
from jax.experimental.pallas import tpu_sc as plsc

N = 10000
E = 320000
D = 128

NC = 2
NS = 16
NW = NC * NS

C = 112
NCH = (156, 24)
NCHUNK_TOT = NS * (NCH[0] + NCH[1])
E_PAD = NCHUNK_TOT * C
N_PAD = N + 8
ROWS_PER_TILE = 624


def _sc_agg_body(with_deg, *refs):
    if with_deg:
        (h_hbm, src_hbm, dst_hbm, z2_hbm, z1_hbm, parts_hbm, degp_hbm,
         s0, s1, s2, s3, s4, s5, d0_, d1_, d2_, d3_, d4_, d5_,
         ones_v, zbuf, acc, deg_acc,
         rows0, rows1, rows2,
         gs0, gs1, gs2, ss0, ss1, ss2,
         is0, is1, is2, is3, is4, is5) = refs
    else:
        (h_hbm, src_hbm, dst_hbm, z2_hbm, parts_hbm,
         s0, s1, s2, s3, s4, s5, d0_, d1_, d2_, d3_, d4_, d5_, acc,
         rows0, rows1, rows2,
         gs0, gs1, gs2, ss0, ss1, ss2,
         is0, is1, is2, is3, is4, is5) = refs
    rows = (rows0, rows1, rows2)
    sidx = (s0, s1, s2, s3, s4, s5)
    didx = (d0_, d1_, d2_, d3_, d4_, d5_)
    gsem = (gs0, gs1, gs2)
    ssem = (ss0, ss1, ss2)
    isem = (is0, is1, is2, is3, is4, is5)

    cid = lax.axis_index("c")
    sid = lax.axis_index("s")
    nchunk = jnp.where(cid == 0, NCH[0], NCH[1])

    r0 = pl.multiple_of(sid * ROWS_PER_TILE, 8)
    pltpu.sync_copy(z2_hbm.at[pl.ds(r0, ROWS_PER_TILE)],
                    acc.at[pl.ds(r0, ROWS_PER_TILE)])

    @pl.when(sid == 0)
    def _zero_rest():
        pltpu.sync_copy(z2_hbm.at[pl.ds(NS * ROWS_PER_TILE, N_PAD - NS * ROWS_PER_TILE)],
                        acc.at[pl.ds(NS * ROWS_PER_TILE, N_PAD - NS * ROWS_PER_TILE)])

    if with_deg:
        pltpu.sync_copy(z1_hbm.at[pl.ds(r0, ROWS_PER_TILE)], zbuf)
        pltpu.sync_copy(zbuf, deg_acc.at[pl.ds(r0, ROWS_PER_TILE)])

        @pl.when(sid == 0)
        def _zero_deg_rest():
            rr = pl.multiple_of(NS * ROWS_PER_TILE, 8)
            nrem = N_PAD - NS * ROWS_PER_TILE
            pltpu.sync_copy(z1_hbm.at[pl.ds(rr, nrem)], zbuf.at[pl.ds(0, nrem)])
            pltpu.sync_copy(zbuf.at[pl.ds(0, nrem)], deg_acc.at[pl.ds(rr, nrem)])

        for k in range(C // 16):
            ones_v[pl.ds(k * 16, 16)] = jnp.ones((16,), jnp.float32)

    plsc.subcore_barrier()

    base = jnp.where(cid == 0, sid * (NCH[0] * C),
                     NS * NCH[0] * C + sid * (NCH[1] * C))

    def load_idx(c, k, sfx):
        off = pl.multiple_of(sfx + c * C, 8)
        pltpu.async_copy(src_hbm.at[pl.ds(off, C)], sidx[k], isem[k])
        pltpu.async_copy(dst_hbm.at[pl.ds(off, C)], didx[k], isem[k])

    def wait_idx(c, k, sfx):
        off = pl.multiple_of(sfx + c * C, 8)
        pltpu.make_async_copy(src_hbm.at[pl.ds(off, C)], sidx[k],
                              isem[k]).wait()
        pltpu.make_async_copy(dst_hbm.at[pl.ds(off, C)], didx[k],
                              isem[k]).wait()

    for k in range(6):
        load_idx(k, k, base)
    for b in range(2):
        wait_idx(b, b, base)
        pltpu.async_copy(h_hbm.at[sidx[b]], rows[b], gsem[b])

    def step6(j, carry):
        c0 = j * 6
        for u in range(6):
            v = u % 3
            c = c0 + u
            pltpu.make_async_copy(h_hbm.at[sidx[u]], rows[v],
                                  gsem[v]).wait()
            pltpu.async_copy(rows[v], acc.at[didx[u]], ssem[v], add=True)
            if with_deg:
                pltpu.sync_copy(ones_v, deg_acc.at[didx[u]], add=True)

            @pl.when(c >= 1)
            def _drain_prev_scatter():
                vp = (v + 2) % 3
                pltpu.make_async_copy(rows[vp], acc.at[didx[(u + 5) % 6]],
                                      ssem[vp]).wait()

            @pl.when((c >= 1) & (c + 5 < nchunk))
            def _prefetch_idx():
                load_idx(c + 5, (u + 5) % 6, base)

            @pl.when(c + 2 < nchunk)
            def _next_gather():
                k2 = (u + 2) % 6
                v2 = (v + 2) % 3
                wait_idx(c + 2, k2, base)
                pltpu.async_copy(h_hbm.at[sidx[k2]], rows[v2], gsem[v2])
        return carry

    lax.fori_loop(0, nchunk // 6, step6, 0)

    pltpu.make_async_copy(rows[2], acc.at[didx[5]], ssem[2]).wait()

    plsc.subcore_barrier()

    pltpu.sync_copy(acc.at[pl.ds(r0, ROWS_PER_TILE)],
                    parts_hbm.at[cid, pl.ds(r0, ROWS_PER_TILE), :])

    @pl.when(sid == 0)
    def _out_rest():
        rr = pl.multiple_of(NS * ROWS_PER_TILE, 8)
        pltpu.sync_copy(acc.at[pl.ds(rr, N - NS * ROWS_PER_TILE)],
                        parts_hbm.at[cid, pl.ds(rr, N - NS * ROWS_PER_TILE), :])

    if with_deg:
        d0 = pl.multiple_of(cid * N + r0, 8)
        pltpu.sync_copy(deg_acc.at[pl.ds(r0, ROWS_PER_TILE)], zbuf)
        pltpu.sync_copy(zbuf, degp_hbm.at[pl.ds(d0, ROWS_PER_TILE)])

        @pl.when(sid == 0)
        def _deg_out_rest():
            rr = pl.multiple_of(NS * ROWS_PER_TILE, 8)
            dd = pl.multiple_of(cid * N + rr, 8)
            nrem = N - NS * ROWS_PER_TILE
            pltpu.sync_copy(deg_acc.at[pl.ds(rr, nrem)], zbuf.at[pl.ds(0, nrem)])
            pltpu.sync_copy(zbuf.at[pl.ds(0, nrem)], degp_hbm.at[pl.ds(dd, nrem)])


def _make_sc_agg(with_deg):
    mesh = plsc.VectorSubcoreMesh(core_axis_name="c", subcore_axis_name="s",
                                  num_cores=NC, num_subcores=NS)
    out_type = [jax.ShapeDtypeStruct((NC, N, D), jnp.float32)]
    if with_deg:
        out_type.append(jax.ShapeDtypeStruct((NC * N,), jnp.float32))
    scratch = [pltpu.VMEM((C,), jnp.int32) for _ in range(12)]
    if with_deg:
        scratch.append(pltpu.VMEM((C,), jnp.float32))
        scratch.append(pltpu.VMEM((ROWS_PER_TILE,), jnp.float32))
    scratch.append(pltpu.VMEM_SHARED((N_PAD, D), jnp.float32))
    if with_deg:
        scratch.append(pltpu.VMEM_SHARED((N_PAD,), jnp.float32))
    scratch.extend(pltpu.VMEM((C, D), jnp.float32) for _ in range(3))
    scratch.extend(pltpu.SemaphoreType.DMA for _ in range(12))
    return pl.kernel(functools.partial(_sc_agg_body, with_deg),
                     out_type=tuple(out_type), mesh=mesh,
                     scratch_types=tuple(scratch))


_sc_agg_first = _make_sc_agg(True)
_sc_agg_rest = _make_sc_agg(False)


def _combine_body(relu, parts_ref, degp_ref, h_ref, wv_ref, b_ref, out_ref):
    agg = parts_ref[0] + parts_ref[1]
    deg = degp_ref[0] + degp_ref[1]
    aggn = agg / jnp.clip(deg, 1.0, None)
    cat = jnp.concatenate([aggn, h_ref[...]], axis=1)
    out = jax.lax.dot_general(cat, wv_ref[...], (((1,), (0,)), ((), ())),
                              preferred_element_type=jnp.float32)
    out = out + b_ref[...]
    if relu:
        out = jnp.maximum(out, 0.0)
    out_ref[...] = out


R = 1000


def _make_combine(relu):
    return pl.pallas_call(
        functools.partial(_combine_body, relu),
        grid=(N // R,),
        in_specs=[
            pl.BlockSpec((NC, R, D), lambda i: (0, i, 0)),
            pl.BlockSpec((NC, R, 1), lambda i: (0, i, 0)),
            pl.BlockSpec((R, D), lambda i: (i, 0)),
            pl.BlockSpec((2 * D, D), lambda i: (0, 0)),
            pl.BlockSpec((1, D), lambda i: (0, 0)),
        ],
        out_specs=pl.BlockSpec((R, D), lambda i: (i, 0)),
        out_shape=jax.ShapeDtypeStruct((N, D), jnp.float32),
    )


_combine_relu = _make_combine(True)
_combine_lin = _make_combine(False)


def kernel(x, edge_index, Wl1, bl1, Wr1, Wl2, bl2, Wr2, Wl3, bl3, Wr3,
           Wl4, bl4, Wr4):
    src = edge_index[0].astype(jnp.int32)
    dst = edge_index[1].astype(jnp.int32)
    pad = E_PAD - E
    src_p = jnp.concatenate([src, jnp.zeros((pad,), jnp.int32)])
    dst_p = jnp.concatenate(
        [dst, N + (jnp.arange(pad, dtype=jnp.int32) % (N_PAD - N))])
    z2 = jnp.zeros((N_PAD, D), jnp.float32)
    z1 = jnp.zeros((N_PAD,), jnp.float32)

    parts, degp = _sc_agg_first(x, src_p, dst_p, z2, z1)
    degp3 = degp.reshape(NC, N, 1)

    layers = [(Wl1, bl1, Wr1, True), (Wl2, bl2, Wr2, False),
              (Wl3, bl3, Wr3, False), (Wl4, bl4, Wr4, False)]

    h = x
    for li, (Wl, bl, Wr, relu) in enumerate(layers):
        if li > 0:
            (parts,) = _sc_agg_rest(h, src_p, dst_p, z2)
        wv = jnp.concatenate([Wl.T, Wr.T], axis=0)
        comb = _combine_relu if relu else _combine_lin
        h = comb(parts, degp3, h, wv, bl.reshape(1, D))
    return h


# speedup vs baseline: 1.4230x; 1.0105x over previous
import functools

import jax
import jax.numpy as jnp
from jax import lax
from jax.experimental import pallas as pl
from jax.experimental.pallas import tpu as pltpu
from jax.experimental.pallas import tpu_sc as plsc

N = 10000
E = 320000
D = 128

NC = 2
NS = 16
NW = NC * NS

C = 112
NCH = (156, 24)
NCHUNK_TOT = NS * (NCH[0] + NCH[1])
E_PAD = NCHUNK_TOT * C
N_PAD = N + 8
ROWS_PER_TILE = 624


def _sc_agg_body(with_deg, *refs):
    if with_deg:
        (h_hbm, src_hbm, dst_hbm, z2_hbm, z1_hbm, parts_hbm, degp_hbm,
         s0, s1, s2, s3, s4, s5, d0_, d1_, d2_, d3_, d4_, d5_,
         ones_v, zbuf, acc, deg_acc,
         rows0, rows1, rows2,
         gs0, gs1, gs2, ss0, ss1, ss2,
         is0, is1, is2, is3, is4, is5) = refs
    else:
        (h_hbm, src_hbm, dst_hbm, z2_hbm, parts_hbm,
         s0, s1, s2, s3, s4, s5, d0_, d1_, d2_, d3_, d4_, d5_, acc,
         rows0, rows1, rows2,
         gs0, gs1, gs2, ss0, ss1, ss2,
         is0, is1, is2, is3, is4, is5) = refs
    rows = (rows0, rows1, rows2)
    sidx = (s0, s1, s2, s3, s4, s5)
    didx = (d0_, d1_, d2_, d3_, d4_, d5_)
    gsem = (gs0, gs1, gs2)
    ssem = (ss0, ss1, ss2)
    isem = (is0, is1, is2, is3, is4, is5)

    cid = lax.axis_index("c")
    sid = lax.axis_index("s")
    nchunk = jnp.where(cid == 0, NCH[0], NCH[1])

    r0 = pl.multiple_of(sid * ROWS_PER_TILE, 8)
    pltpu.sync_copy(z2_hbm.at[pl.ds(r0, ROWS_PER_TILE)],
                    acc.at[pl.ds(r0, ROWS_PER_TILE)])

    @pl.when(sid == 0)
    def _zero_rest():
        pltpu.sync_copy(z2_hbm.at[pl.ds(NS * ROWS_PER_TILE, N_PAD - NS * ROWS_PER_TILE)],
                        acc.at[pl.ds(NS * ROWS_PER_TILE, N_PAD - NS * ROWS_PER_TILE)])

    if with_deg:
        pltpu.sync_copy(z1_hbm.at[pl.ds(r0, ROWS_PER_TILE)], zbuf)
        pltpu.sync_copy(zbuf, deg_acc.at[pl.ds(r0, ROWS_PER_TILE)])

        @pl.when(sid == 0)
        def _zero_deg_rest():
            rr = pl.multiple_of(NS * ROWS_PER_TILE, 8)
            nrem = N_PAD - NS * ROWS_PER_TILE
            pltpu.sync_copy(z1_hbm.at[pl.ds(rr, nrem)], zbuf.at[pl.ds(0, nrem)])
            pltpu.sync_copy(zbuf.at[pl.ds(0, nrem)], deg_acc.at[pl.ds(rr, nrem)])

        for k in range(C // 16):
            ones_v[pl.ds(k * 16, 16)] = jnp.ones((16,), jnp.float32)

    plsc.subcore_barrier()

    base = jnp.where(cid == 0, sid * (NCH[0] * C),
                     NS * NCH[0] * C + sid * (NCH[1] * C))

    def load_idx(c, k, sfx):
        off = pl.multiple_of(sfx + c * C, 8)
        pltpu.async_copy(src_hbm.at[pl.ds(off, C)], sidx[k], isem[k])
        pltpu.async_copy(dst_hbm.at[pl.ds(off, C)], didx[k], isem[k])

    def wait_idx(c, k, sfx):
        off = pl.multiple_of(sfx + c * C, 8)
        pltpu.make_async_copy(src_hbm.at[pl.ds(off, C)], sidx[k],
                              isem[k]).wait()
        pltpu.make_async_copy(dst_hbm.at[pl.ds(off, C)], didx[k],
                              isem[k]).wait()

    for k in range(6):
        load_idx(k, k, base)
    for b in range(2):
        wait_idx(b, b, base)
        pltpu.async_copy(h_hbm.at[sidx[b]], rows[b], gsem[b])

    def step6(j, carry):
        c0 = j * 6
        for u in range(6):
            v = u % 3
            c = c0 + u
            pltpu.make_async_copy(h_hbm.at[sidx[u]], rows[v],
                                  gsem[v]).wait()
            pltpu.async_copy(rows[v], acc.at[didx[u]], ssem[v], add=True)
            if with_deg:
                pltpu.sync_copy(ones_v, deg_acc.at[didx[u]], add=True)

            @pl.when(c >= 1)
            def _drain_prev_scatter():
                vp = (v + 2) % 3
                pltpu.make_async_copy(rows[vp], acc.at[didx[(u + 5) % 6]],
                                      ssem[vp]).wait()

            @pl.when((c >= 1) & (c + 5 < nchunk))
            def _prefetch_idx():
                load_idx(c + 5, (u + 5) % 6, base)

            @pl.when(c + 2 < nchunk)
            def _next_gather():
                k2 = (u + 2) % 6
                v2 = (v + 2) % 3
                wait_idx(c + 2, k2, base)
                pltpu.async_copy(h_hbm.at[sidx[k2]], rows[v2], gsem[v2])
        return carry

    lax.fori_loop(0, nchunk // 6, step6, 0)

    pltpu.make_async_copy(rows[2], acc.at[didx[5]], ssem[2]).wait()

    plsc.subcore_barrier()

    pltpu.sync_copy(acc.at[pl.ds(r0, ROWS_PER_TILE)],
                    parts_hbm.at[cid, pl.ds(r0, ROWS_PER_TILE), :])

    @pl.when(sid == 0)
    def _out_rest():
        rr = pl.multiple_of(NS * ROWS_PER_TILE, 8)
        pltpu.sync_copy(acc.at[pl.ds(rr, N - NS * ROWS_PER_TILE)],
                        parts_hbm.at[cid, pl.ds(rr, N - NS * ROWS_PER_TILE), :])

    if with_deg:
        d0 = pl.multiple_of(cid * N + r0, 8)
        pltpu.sync_copy(deg_acc.at[pl.ds(r0, ROWS_PER_TILE)], zbuf)
        pltpu.sync_copy(zbuf, degp_hbm.at[pl.ds(d0, ROWS_PER_TILE)])

        @pl.when(sid == 0)
        def _deg_out_rest():
            rr = pl.multiple_of(NS * ROWS_PER_TILE, 8)
            dd = pl.multiple_of(cid * N + rr, 8)
            nrem = N - NS * ROWS_PER_TILE
            pltpu.sync_copy(deg_acc.at[pl.ds(rr, nrem)], zbuf.at[pl.ds(0, nrem)])
            pltpu.sync_copy(zbuf.at[pl.ds(0, nrem)], degp_hbm.at[pl.ds(dd, nrem)])


def _make_sc_agg(with_deg):
    mesh = plsc.VectorSubcoreMesh(core_axis_name="c", subcore_axis_name="s",
                                  num_cores=NC, num_subcores=NS)
    out_type = [jax.ShapeDtypeStruct((NC, N, D), jnp.float32)]
    if with_deg:
        out_type.append(jax.ShapeDtypeStruct((NC * N,), jnp.float32))
    scratch = [pltpu.VMEM((C,), jnp.int32) for _ in range(12)]
    if with_deg:
        scratch.append(pltpu.VMEM((C,), jnp.float32))
        scratch.append(pltpu.VMEM((ROWS_PER_TILE,), jnp.float32))
    scratch.append(pltpu.VMEM_SHARED((N_PAD, D), jnp.float32))
    if with_deg:
        scratch.append(pltpu.VMEM_SHARED((N_PAD,), jnp.float32))
    scratch.extend(pltpu.VMEM((C, D), jnp.float32) for _ in range(3))
    scratch.extend(pltpu.SemaphoreType.DMA for _ in range(12))
    return pl.kernel(functools.partial(_sc_agg_body, with_deg),
                     out_type=tuple(out_type), mesh=mesh,
                     scratch_types=tuple(scratch))


_sc_agg_first = _make_sc_agg(True)
_sc_agg_rest = _make_sc_agg(False)


def _combine_body(relu, parts_ref, degp_ref, h_ref, wv_ref, b_ref, out_ref):
    agg = parts_ref[0] + parts_ref[1]
    deg = degp_ref[0] + degp_ref[1]
    aggn = agg / jnp.clip(deg, 1.0, None)
    cat = jnp.concatenate([aggn, h_ref[...]], axis=1)
    out = jax.lax.dot_general(cat, wv_ref[...], (((1,), (0,)), ((), ())),
                              preferred_element_type=jnp.float32)
    out = out + b_ref[...]
    if relu:
        out = jnp.maximum(out, 0.0)
    out_ref[...] = out


R = 2000


def _make_combine(relu):
    return pl.pallas_call(
        functools.partial(_combine_body, relu),
        grid=(N // R,),
        in_specs=[
            pl.BlockSpec((NC, R, D), lambda i: (0, i, 0)),
            pl.BlockSpec((NC, R, 1), lambda i: (0, i, 0)),
            pl.BlockSpec((R, D), lambda i: (i, 0)),
            pl.BlockSpec((2 * D, D), lambda i: (0, 0)),
            pl.BlockSpec((1, D), lambda i: (0, 0)),
        ],
        out_specs=pl.BlockSpec((R, D), lambda i: (i, 0)),
        out_shape=jax.ShapeDtypeStruct((N, D), jnp.float32),
    )


_combine_relu = _make_combine(True)
_combine_lin = _make_combine(False)


def kernel(x, edge_index, Wl1, bl1, Wr1, Wl2, bl2, Wr2, Wl3, bl3, Wr3,
           Wl4, bl4, Wr4):
    src = edge_index[0].astype(jnp.int32)
    dst = edge_index[1].astype(jnp.int32)
    pad = E_PAD - E
    src_p = jnp.concatenate([src, jnp.zeros((pad,), jnp.int32)])
    dst_p = jnp.concatenate(
        [dst, N + (jnp.arange(pad, dtype=jnp.int32) % (N_PAD - N))])
    z2 = jnp.zeros((N_PAD, D), jnp.float32)
    z1 = jnp.zeros((N_PAD,), jnp.float32)

    parts, degp = _sc_agg_first(x, src_p, dst_p, z2, z1)
    degp3 = degp.reshape(NC, N, 1)

    layers = [(Wl1, bl1, Wr1, True), (Wl2, bl2, Wr2, False),
              (Wl3, bl3, Wr3, False), (Wl4, bl4, Wr4, False)]

    h = x
    for li, (Wl, bl, Wr, relu) in enumerate(layers):
        if li > 0:
            (parts,) = _sc_agg_rest(h, src_p, dst_p, z2)
        wv = jnp.concatenate([Wl.T, Wr.T], axis=0)
        comb = _combine_relu if relu else _combine_lin
        h = comb(parts, degp3, h, wv, bl.reshape(1, D))
    return h
